# baseline (speedup 1.0000x reference)
import functools

import jax
import jax.numpy as jnp
from jax import lax
from jax.experimental import pallas as pl
from jax.experimental.pallas import tpu as pltpu
from jax.experimental.pallas import tpu_sc as plsc

ITEMS = 100000
DIM = 32
BATCH = 4
SEG = 512
NUPD = BATCH * SEG
NWRK = 32
RPW = NUPD // NWRK
AGRP = (ITEMS + 127) // 128
NBLK = 50
RPB = ITEMS // NBLK


def _sc_body(ids_hbm, pos_hbm, feat_hbm, emb_hbm, alphap_hbm, val_hbm,
             idsv, posv, grpv, emb_v, feat_v, alphag_v, val_v, gsem):
    c = lax.axis_index("c")
    s = lax.axis_index("s")
    wid = s * 2 + c
    u0 = pl.multiple_of(wid * RPW, 8)

    pltpu.sync_copy(ids_hbm.at[pl.ds(u0, RPW)], idsv)
    pltpu.sync_copy(pos_hbm.at[pl.ds(u0, RPW)], posv)

    for g in range(RPW // 16):
        sl = pl.ds(g * 16, 16)
        grpv[sl] = idsv[sl] // 128

    d1 = pltpu.async_copy(emb_hbm.at[idsv], emb_v, gsem)
    d2 = pltpu.async_copy(feat_hbm.at[posv], feat_v, gsem)
    d3 = pltpu.async_copy(alphap_hbm.at[grpv], alphag_v, gsem)
    d1.wait()
    d2.wait()
    d3.wait()

    lanes = lax.broadcasted_iota(jnp.int32, (16,), 0)
    for g in range(RPW // 16):
        u16 = g * 16 + lanes
        ids16 = idsv[pl.ds(g * 16, 16)]
        a16 = plsc.load_gather(alphag_v, [u16, ids16 % 128])
        for k in range(16):
            u = g * 16 + k
            a = a16[k]
            for h in range(DIM // 16):
                e = emb_v[u, pl.ds(h * 16, 16)]
                f = feat_v[u, pl.ds(h * 16, 16)]
                val_v[u, pl.ds(h * 16, 16)] = e + a * (f - e)

    pltpu.sync_copy(val_v, val_hbm.at[pl.ds(u0, RPW)])


def _sc_values(ids_s, pos_s, feat, emb, alphap):
    mesh = plsc.VectorSubcoreMesh(core_axis_name="c", subcore_axis_name="s")
    f = functools.partial(
        pl.kernel,
        out_type=jax.ShapeDtypeStruct((NUPD, DIM), jnp.float32),
        mesh=mesh,
        compiler_params=pltpu.CompilerParams(needs_layout_passes=False,
                                             use_tc_tiling_on_sc=False),
        scratch_types=[
            pltpu.VMEM((RPW,), jnp.int32),
            pltpu.VMEM((RPW,), jnp.int32),
            pltpu.VMEM((RPW,), jnp.int32),
            pltpu.VMEM((RPW, DIM), jnp.float32),
            pltpu.VMEM((RPW, DIM), jnp.float32),
            pltpu.VMEM((RPW, 128), jnp.float32),
            pltpu.VMEM((RPW, DIM), jnp.float32),
            pltpu.SemaphoreType.DMA,
        ],
    )(_sc_body)
    return f(ids_s, pos_s, feat, emb, alphap)


def _tc_body(starts_ref, ids_ref, emb_ref, val_ref, out_ref):
    i = pl.program_id(0)
    for b in range(BATCH):
        out_ref[b, :, :] = emb_ref[:, :]
    for b in range(BATCH):
        s0 = starts_ref[b, i]
        s1 = starts_ref[b, i + 1]

        def ubody(j, carry, b=b):
            rr = ids_ref[b, j] - i * RPB
            out_ref[b, pl.ds(rr, 1), :] = val_ref[b, pl.ds(j, 1), :]
            return carry

        lax.fori_loop(s0, s1, ubody, jnp.int32(0))


def _tc_assemble(starts, ids_s, emb, val3):
    grid_spec = pltpu.PrefetchScalarGridSpec(
        num_scalar_prefetch=2,
        grid=(NBLK,),
        in_specs=[
            pl.BlockSpec((RPB, DIM), lambda i, *_: (i, 0)),
            pl.BlockSpec((BATCH, SEG, DIM), lambda i, *_: (0, 0, 0)),
        ],
        out_specs=pl.BlockSpec((BATCH, RPB, DIM), lambda i, *_: (0, i, 0)),
    )
    return pl.pallas_call(
        _tc_body,
        grid_spec=grid_spec,
        out_shape=jax.ShapeDtypeStruct((BATCH, ITEMS, DIM), jnp.float32),
    )(starts, ids_s, emb, val3)


def kernel(ptr, nodes, nodes_output, emb_table, alpha):
    del ptr
    ids2 = nodes.reshape(BATCH, SEG)
    order = jnp.argsort(ids2, axis=1, stable=True)
    ids_s = jnp.take_along_axis(ids2, order, axis=1)
    pos_s = order + (jnp.arange(BATCH, dtype=jnp.int32) * SEG)[:, None]
    bounds = jnp.arange(NBLK + 1, dtype=jnp.int32) * RPB
    starts = jax.vmap(
        lambda r: jnp.searchsorted(r, bounds, side="left")
    )(ids_s).astype(jnp.int32)

    alphap = jnp.concatenate(
        [alpha[:, 0], jnp.zeros((AGRP * 128 - ITEMS,), jnp.float32)]
    ).reshape(AGRP, 128)

    val = _sc_values(ids_s.reshape(-1), pos_s.reshape(-1), nodes_output,
                     emb_table, alphap)
    return _tc_assemble(starts, ids_s, emb_table,
                        val.reshape(BATCH, SEG, DIM))


# speedup vs baseline: 1.0602x; 1.0602x over previous
import functools

import jax
import jax.numpy as jnp
from jax import lax
from jax.experimental import pallas as pl
from jax.experimental.pallas import tpu as pltpu
from jax.experimental.pallas import tpu_sc as plsc

ITEMS = 100000
DIM = 32
BATCH = 4
SEG = 512
NUPD = BATCH * SEG
NWRK = 32
RPW = NUPD // NWRK
AGRP = (ITEMS + 127) // 128
NBLK = 20
RPB = ITEMS // NBLK


def _sc_body(ids_hbm, pos_hbm, feat_hbm, emb_hbm, alphap_hbm, val_hbm,
             idsv, posv, grpv, emb_v, feat_v, alphag_v, val_v, gsem):
    c = lax.axis_index("c")
    s = lax.axis_index("s")
    wid = s * 2 + c
    u0 = pl.multiple_of(wid * RPW, 8)

    pltpu.sync_copy(ids_hbm.at[pl.ds(u0, RPW)], idsv)
    pltpu.sync_copy(pos_hbm.at[pl.ds(u0, RPW)], posv)

    for g in range(RPW // 16):
        sl = pl.ds(g * 16, 16)
        grpv[sl] = idsv[sl] // 128

    d1 = pltpu.async_copy(emb_hbm.at[idsv], emb_v, gsem)
    d2 = pltpu.async_copy(feat_hbm.at[posv], feat_v, gsem)
    d3 = pltpu.async_copy(alphap_hbm.at[grpv], alphag_v, gsem)
    d1.wait()
    d2.wait()
    d3.wait()

    lanes = lax.broadcasted_iota(jnp.int32, (16,), 0)
    for g in range(RPW // 16):
        u16 = g * 16 + lanes
        ids16 = idsv[pl.ds(g * 16, 16)]
        a16 = plsc.load_gather(alphag_v, [u16, ids16 % 128])
        for k in range(16):
            u = g * 16 + k
            a = a16[k]
            for h in range(DIM // 16):
                e = emb_v[u, pl.ds(h * 16, 16)]
                f = feat_v[u, pl.ds(h * 16, 16)]
                val_v[u, pl.ds(h * 16, 16)] = e + a * (f - e)

    pltpu.sync_copy(val_v, val_hbm.at[pl.ds(u0, RPW)])


def _sc_values(ids_s, pos_s, feat, emb, alphap):
    mesh = plsc.VectorSubcoreMesh(core_axis_name="c", subcore_axis_name="s")
    f = functools.partial(
        pl.kernel,
        out_type=jax.ShapeDtypeStruct((NUPD, DIM), jnp.float32),
        mesh=mesh,
        compiler_params=pltpu.CompilerParams(needs_layout_passes=False,
                                             use_tc_tiling_on_sc=False),
        scratch_types=[
            pltpu.VMEM((RPW,), jnp.int32),
            pltpu.VMEM((RPW,), jnp.int32),
            pltpu.VMEM((RPW,), jnp.int32),
            pltpu.VMEM((RPW, DIM), jnp.float32),
            pltpu.VMEM((RPW, DIM), jnp.float32),
            pltpu.VMEM((RPW, 128), jnp.float32),
            pltpu.VMEM((RPW, DIM), jnp.float32),
            pltpu.SemaphoreType.DMA,
        ],
    )(_sc_body)
    return f(ids_s, pos_s, feat, emb, alphap)


def _tc_body(starts_ref, ids_ref, emb_ref, val_ref, out_ref):
    i = pl.program_id(0)
    for b in range(BATCH):
        out_ref[b, :, :] = emb_ref[:, :]
    for b in range(BATCH):
        s0 = starts_ref[b, i]
        s1 = starts_ref[b, i + 1]

        def ubody(j, carry, b=b):
            rr = ids_ref[b, j] - i * RPB
            out_ref[b, pl.ds(rr, 1), :] = val_ref[b, pl.ds(j, 1), :]
            return carry

        lax.fori_loop(s0, s1, ubody, jnp.int32(0))


def _tc_assemble(starts, ids_s, emb, val3):
    grid_spec = pltpu.PrefetchScalarGridSpec(
        num_scalar_prefetch=2,
        grid=(NBLK,),
        in_specs=[
            pl.BlockSpec((RPB, DIM), lambda i, *_: (i, 0)),
            pl.BlockSpec((BATCH, SEG, DIM), lambda i, *_: (0, 0, 0)),
        ],
        out_specs=pl.BlockSpec((BATCH, RPB, DIM), lambda i, *_: (0, i, 0)),
    )
    return pl.pallas_call(
        _tc_body,
        grid_spec=grid_spec,
        out_shape=jax.ShapeDtypeStruct((BATCH, ITEMS, DIM), jnp.float32),
    )(starts, ids_s, emb, val3)


def kernel(ptr, nodes, nodes_output, emb_table, alpha):
    del ptr
    ids2 = nodes.reshape(BATCH, SEG)
    order = jnp.argsort(ids2, axis=1, stable=True)
    ids_s = jnp.take_along_axis(ids2, order, axis=1)
    pos_s = order + (jnp.arange(BATCH, dtype=jnp.int32) * SEG)[:, None]
    bounds = jnp.arange(NBLK + 1, dtype=jnp.int32) * RPB
    starts = jax.vmap(
        lambda r: jnp.searchsorted(r, bounds, side="left")
    )(ids_s).astype(jnp.int32)

    alphap = jnp.concatenate(
        [alpha[:, 0], jnp.zeros((AGRP * 128 - ITEMS,), jnp.float32)]
    ).reshape(AGRP, 128)

    val = _sc_values(ids_s.reshape(-1), pos_s.reshape(-1), nodes_output,
                     emb_table, alphap)
    return _tc_assemble(starts, ids_s, emb_table,
                        val.reshape(BATCH, SEG, DIM))


# speedup vs baseline: 1.1040x; 1.0413x over previous
import functools

import jax
import jax.numpy as jnp
from jax import lax
from jax.experimental import pallas as pl
from jax.experimental.pallas import tpu as pltpu
from jax.experimental.pallas import tpu_sc as plsc

ITEMS = 100000
DIM = 32
BATCH = 4
SEG = 512
NUPD = BATCH * SEG
NWRK = 32
RPW = NUPD // NWRK
AGRP = (ITEMS + 127) // 128
NBLK = 10
RPB = ITEMS // NBLK


def _sc_body(ids_hbm, pos_hbm, feat_hbm, emb_hbm, alphap_hbm, val_hbm,
             idsv, posv, grpv, emb_v, feat_v, alphag_v, val_v, gsem):
    c = lax.axis_index("c")
    s = lax.axis_index("s")
    wid = s * 2 + c
    u0 = pl.multiple_of(wid * RPW, 8)

    pltpu.sync_copy(ids_hbm.at[pl.ds(u0, RPW)], idsv)
    pltpu.sync_copy(pos_hbm.at[pl.ds(u0, RPW)], posv)

    for g in range(RPW // 16):
        sl = pl.ds(g * 16, 16)
        grpv[sl] = idsv[sl] // 128

    d1 = pltpu.async_copy(emb_hbm.at[idsv], emb_v, gsem)
    d2 = pltpu.async_copy(feat_hbm.at[posv], feat_v, gsem)
    d3 = pltpu.async_copy(alphap_hbm.at[grpv], alphag_v, gsem)
    d1.wait()
    d2.wait()
    d3.wait()

    lanes = lax.broadcasted_iota(jnp.int32, (16,), 0)
    for g in range(RPW // 16):
        u16 = g * 16 + lanes
        ids16 = idsv[pl.ds(g * 16, 16)]
        a16 = plsc.load_gather(alphag_v, [u16, ids16 % 128])
        for k in range(16):
            u = g * 16 + k
            a = a16[k]
            for h in range(DIM // 16):
                e = emb_v[u, pl.ds(h * 16, 16)]
                f = feat_v[u, pl.ds(h * 16, 16)]
                val_v[u, pl.ds(h * 16, 16)] = e + a * (f - e)

    pltpu.sync_copy(val_v, val_hbm.at[pl.ds(u0, RPW)])


def _sc_values(ids_s, pos_s, feat, emb, alphap):
    mesh = plsc.VectorSubcoreMesh(core_axis_name="c", subcore_axis_name="s")
    f = functools.partial(
        pl.kernel,
        out_type=jax.ShapeDtypeStruct((NUPD, DIM), jnp.float32),
        mesh=mesh,
        compiler_params=pltpu.CompilerParams(needs_layout_passes=False,
                                             use_tc_tiling_on_sc=False),
        scratch_types=[
            pltpu.VMEM((RPW,), jnp.int32),
            pltpu.VMEM((RPW,), jnp.int32),
            pltpu.VMEM((RPW,), jnp.int32),
            pltpu.VMEM((RPW, DIM), jnp.float32),
            pltpu.VMEM((RPW, DIM), jnp.float32),
            pltpu.VMEM((RPW, 128), jnp.float32),
            pltpu.VMEM((RPW, DIM), jnp.float32),
            pltpu.SemaphoreType.DMA,
        ],
    )(_sc_body)
    return f(ids_s, pos_s, feat, emb, alphap)


def _tc_body(starts_ref, ids_ref, emb_ref, val_ref, out_ref):
    i = pl.program_id(0)
    for b in range(BATCH):
        out_ref[b, :, :] = emb_ref[:, :]
    for b in range(BATCH):
        s0 = starts_ref[b, i]
        s1 = starts_ref[b, i + 1]

        def ubody(j, carry, b=b):
            rr = ids_ref[b, j] - i * RPB
            out_ref[b, pl.ds(rr, 1), :] = val_ref[b, pl.ds(j, 1), :]
            return carry

        lax.fori_loop(s0, s1, ubody, jnp.int32(0))


def _tc_assemble(starts, ids_s, emb, val3):
    grid_spec = pltpu.PrefetchScalarGridSpec(
        num_scalar_prefetch=2,
        grid=(NBLK,),
        in_specs=[
            pl.BlockSpec((RPB, DIM), lambda i, *_: (i, 0)),
            pl.BlockSpec((BATCH, SEG, DIM), lambda i, *_: (0, 0, 0)),
        ],
        out_specs=pl.BlockSpec((BATCH, RPB, DIM), lambda i, *_: (0, i, 0)),
    )
    return pl.pallas_call(
        _tc_body,
        grid_spec=grid_spec,
        out_shape=jax.ShapeDtypeStruct((BATCH, ITEMS, DIM), jnp.float32),
    )(starts, ids_s, emb, val3)


def kernel(ptr, nodes, nodes_output, emb_table, alpha):
    del ptr
    ids2 = nodes.reshape(BATCH, SEG)
    order = jnp.argsort(ids2, axis=1, stable=True)
    ids_s = jnp.take_along_axis(ids2, order, axis=1)
    pos_s = order + (jnp.arange(BATCH, dtype=jnp.int32) * SEG)[:, None]
    bounds = jnp.arange(NBLK + 1, dtype=jnp.int32) * RPB
    starts = jax.vmap(
        lambda r: jnp.searchsorted(r, bounds, side="left")
    )(ids_s).astype(jnp.int32)

    alphap = jnp.concatenate(
        [alpha[:, 0], jnp.zeros((AGRP * 128 - ITEMS,), jnp.float32)]
    ).reshape(AGRP, 128)

    val = _sc_values(ids_s.reshape(-1), pos_s.reshape(-1), nodes_output,
                     emb_table, alphap)
    return _tc_assemble(starts, ids_s, emb_table,
                        val.reshape(BATCH, SEG, DIM))
